# baseline (speedup 1.0000x reference)
import functools
import jax
import jax.numpy as jnp
from jax.experimental import pallas as pl
from jax.experimental.pallas import tpu as pltpu

N_NODES, E_EDGES, R_RELS, A_ATTRS, EA_EDGES = 10000, 320000, 500, 2000, 200000
D, RD, AD = 128, 64, 64
L, H = 2, 4
HD = D // H


def _leaky(x):
    return jnp.where(x >= 0, x, 0.3 * x)


def _matmul_kernel(x_ref, w_ref, o_ref):
    o_ref[...] = jnp.dot(x_ref[...], w_ref[...],
                         preferred_element_type=jnp.float32)


def _matmul(x, w, block_rows=1024):
    n, k = x.shape
    m = w.shape[1]
    n_pad = ((n + block_rows - 1) // block_rows) * block_rows
    if n_pad != n:
        x = jnp.pad(x, ((0, n_pad - n), (0, 0)))
    out = pl.pallas_call(
        _matmul_kernel,
        grid=(n_pad // block_rows,),
        in_specs=[pl.BlockSpec((block_rows, k), lambda i: (i, 0)),
                  pl.BlockSpec((k, m), lambda i: (0, 0))],
        out_specs=pl.BlockSpec((block_rows, m), lambda i: (i, 0)),
        out_shape=jax.ShapeDtypeStruct((n_pad, m), jnp.float32),
    )(x, w)
    return out[:n]


def _sort_pairs(a, mult):
    key = a[:, 0] * mult + a[:, 1]
    return a[jnp.argsort(key)]


def _unique_sorted(a_sorted, mult, oob):
    key = a_sorted[:, 0] * mult + a_sorted[:, 1]
    n = key.shape[0]
    valid = jnp.concatenate([jnp.ones((1,), bool), key[1:] != key[:-1]])
    inv = jnp.cumsum(valid) - 1
    uk = jnp.full((n,), oob * mult, key.dtype).at[inv].set(key)
    rows = jnp.stack([uk // mult, uk % mult], axis=1)
    return rows, inv


def _soft_agg(seg, valid, logits, vals, n_rows):
    e = jnp.where(valid, jnp.exp(logits), 0.0)
    num = jax.ops.segment_sum(e[:, None] * vals, seg, num_segments=n_rows)
    den = jax.ops.segment_sum(e, seg, num_segments=n_rows)
    return jnp.where(den[:, None] > 0, num / den[:, None], 0.0)


def kernel(ent_emb, rel_emb, attr_emb, wr_w, wr_b, wa_w, wa_b,
           ent_attn_w, ent_attn_b, concept_attn_w, concept_attn_b,
           all_matix, attr_matrix):
    n = ent_emb.shape[0]

    er = _sort_pairs(all_matix[:, 3:5], R_RELS)
    rel_index, _ = _unique_sorted(er, R_RELS, n)
    ea = _sort_pairs(attr_matrix[:, 0:2], A_ATTRS)
    attr_index, _ = _unique_sorted(ea, A_ATTRS, n)
    ee = _sort_pairs(all_matix[:, 0:2], n)
    index, idx = _unique_sorted(ee, n, n)
    uvalid = jnp.zeros((ee.shape[0],), bool).at[idx].set(True)
    src, dst = index[:, 0], index[:, 1]
    valid_c = src < n

    se = _matmul(ent_emb, wr_w[:D]).reshape(-1) + wr_b[0]
    sr = _matmul(rel_emb, wr_w[D:]).reshape(-1)
    sa_e = _matmul(ent_emb, wa_w[:D]).reshape(-1) + wa_b[0]
    sa = _matmul(attr_emb, wa_w[D:]).reshape(-1)

    w_rf_e = jnp.stack([ent_attn_w[l, h, HD:HD + RD, 0]
                        for l in range(L) for h in range(H)], axis=1)
    w_rf_c = jnp.stack([concept_attn_w[l, h, RD + AD:RD + AD + RD, 0]
                        for l in range(L) for h in range(H)], axis=1)
    rsc16 = _matmul(rel_emb, jnp.concatenate([w_rf_e, w_rf_c], axis=1))

    ai, ar = rel_index[:, 0], rel_index[:, 1]
    va = ai < n
    log_a = _leaky(se[jnp.minimum(ai, n - 1)] + sr[ar])
    concept_rel = jax.nn.relu(_soft_agg(ai, va, log_a, rel_emb[ar], n))

    bi, ba = attr_index[:, 0], attr_index[:, 1]
    vb = bi < n
    log_b = _leaky(sa_e[jnp.minimum(bi, n - 1)] + sa[ba])
    concept_attr = jax.nn.relu(_soft_agg(bi, vb, log_b, attr_emb[ba], n))

    deg = jax.ops.segment_sum(jnp.ones_like(src, jnp.float32), src,
                              num_segments=n)
    invdeg = jnp.where(deg > 0, 1.0 / deg, 0.0)
    w0 = jnp.where(valid_c, invdeg[jnp.minimum(src, n - 1)], 0.0)
    x = jax.ops.segment_sum(w0[:, None] * ent_emb[jnp.minimum(dst, n - 1)],
                            src, num_segments=n)

    q = rsc16[er[:, 1]]
    counts = jax.ops.segment_sum(jnp.ones((idx.shape[0],), jnp.float32), idx,
                                 num_segments=E_EDGES)
    rf = (jax.ops.segment_sum(q, idx, num_segments=E_EDGES)
          / jnp.maximum(counts, 1.0)[:, None])

    concept_cat = jnp.concatenate([concept_rel, concept_attr], axis=1)
    w_c1 = jnp.stack([concept_attn_w[l, h, :RD + AD, 0]
                      for l in range(L) for h in range(H)], axis=1)
    w_c2 = jnp.stack([concept_attn_w[l, h, RD + AD + RD:, 0]
                      for l in range(L) for h in range(H)], axis=1)
    c1 = _matmul(concept_cat, w_c1) + concept_attn_b.reshape(1, L * H)
    c2 = _matmul(concept_cat, w_c2)

    src_cl = jnp.minimum(src, n - 1)
    dst_cl = jnp.minimum(dst, n - 1)
    outputs = []
    for l in range(L):
        xr = jax.nn.relu(x)
        w_p1 = jnp.stack([ent_attn_w[l, h, :HD, 0] for h in range(H)], axis=1)
        w_p3 = jnp.stack([ent_attn_w[l, h, HD + RD:, 0] for h in range(H)],
                         axis=1)
        p1 = jnp.stack(
            [xr[:, h * HD:(h + 1) * HD] @ w_p1[:, h] for h in range(H)],
            axis=1) + ent_attn_b[l, :, 0][None, :]
        p3 = jnp.stack(
            [xr[:, h * HD:(h + 1) * HD] @ w_p3[:, h] for h in range(H)],
            axis=1)

        rf_e = rf[:, l * H:(l + 1) * H]
        rf_c = rf[:, 8 + l * H:8 + (l + 1) * H]
        e_attn = _leaky(p1[src_cl] + rf_e + p3[dst_cl])
        c_attn = _leaky(c1[src_cl][:, l * H:(l + 1) * H] + rf_c
                        + c2[dst_cl][:, l * H:(l + 1) * H])
        g = e_attn * c_attn

        s_glob = jnp.sum(jnp.where(valid_c[:, None], jnp.exp(g), 0.0), axis=0)
        a1 = jnp.exp(g) / s_glob[None, :]
        head_feats = []
        for h in range(H):
            he_dst = xr[dst_cl, h * HD:(h + 1) * HD]
            head_feats.append(
                _soft_agg(src, valid_c, a1[:, h], he_dst, n))
        x = jnp.tanh(jnp.concatenate(head_feats, axis=1))
        outputs.append(x)
    return jnp.concatenate(outputs, axis=1)


# speedup vs baseline: 7.3760x; 7.3760x over previous
import dataclasses
import functools
import jax
import jax.numpy as jnp
from jax import lax
from jax.experimental import pallas as pl
from jax.experimental.pallas import tpu as pltpu
from jax.experimental.pallas import tpu_sc as plsc

N_NODES, E_EDGES, R_RELS, A_ATTRS, EA_EDGES = 10000, 320000, 500, 2000, 200000
D, RD, AD = 128, 64, 64
L, H = 2, 4
HD = D // H

NC, NS, LN = 2, 16, 16
NW = NC * NS
NA = 10240
SLOP = N_NODES


def _leaky(x):
    return jnp.where(x >= 0, x, 0.3 * x)


def _iota16():
    return lax.iota(jnp.int32, 16)


def _full16(v):
    return jnp.full((16,), v, jnp.int32)


def _matmul_kernel(x_ref, w_ref, o_ref):
    o_ref[...] = jnp.dot(x_ref[...], w_ref[...],
                         preferred_element_type=jnp.float32)


def _matmul(x, w, block_rows=1024):
    n, k = x.shape
    m = w.shape[1]
    n_pad = ((n + block_rows - 1) // block_rows) * block_rows
    if n_pad != n:
        x = jnp.pad(x, ((0, n_pad - n), (0, 0)))
    out = pl.pallas_call(
        _matmul_kernel,
        grid=(n_pad // block_rows,),
        in_specs=[pl.BlockSpec((block_rows, k), lambda i: (i, 0)),
                  pl.BlockSpec((k, m), lambda i: (0, 0))],
        out_specs=pl.BlockSpec((block_rows, m), lambda i: (i, 0)),
        out_shape=jax.ShapeDtypeStruct((n_pad, m), jnp.float32),
    )(x, w)
    return out[:n]


_MESH = plsc.VectorSubcoreMesh(core_axis_name="c", subcore_axis_name="s")
_CP = pltpu.CompilerParams()
if "needs_layout_passes" in pltpu.CompilerParams.__dataclass_fields__:
    _CP = dataclasses.replace(_CP, needs_layout_passes=False,
                              use_tc_tiling_on_sc=False)


def _zero_acc(acc, zbuf, wa, sub):
    for c in range(wa // 16):
        zbuf[0, pl.ds(c * 16, 16)] = jnp.zeros((16,), jnp.float32)
    for r in range(1, 16):
        for c in range(wa // 16):
            zbuf[r, pl.ds(c * 16, 16)] = zbuf[0, pl.ds(c * 16, 16)]
    stripe = NA // NS

    @pl.loop(0, stripe // 16)
    def _(k):
        pltpu.sync_copy(zbuf, acc.at[pl.ds(sub * stripe + k * 16, 16)])


def _copy_out(acc, out, core, sub):
    stripe = NA // NS
    pltpu.sync_copy(acc.at[pl.ds(sub * stripe, stripe)],
                    out.at[core, pl.ds(sub * stripe, stripe)])


def _scale_rows(rows, stg, wbuf, wg, wcol):
    for i in range(16):
        ws = plsc.load_gather(wbuf, [_full16(i)])
        for c in range(wg // 16):
            stg[i, pl.ds(c * 16, 16)] = rows[i, pl.ds(c * 16, 16)] * ws
        if wcol:
            stg[i, pl.ds(wg, 16)] = jnp.where(_iota16() == 0, ws, 0.0)


def _sc_soft_agg(sv, tv, table, stab, ttab, ep, blk, mode):
    wg = table.shape[1]
    wa = wg + 16 if mode == "exp" else wg
    per_w = ep // NW
    nblk = per_w // blk
    nstab = stab.shape[0]
    nttab = ttab.shape[0] if ttab is not None else 0

    def body(sv_hbm, tv_hbm, tab_hbm, stab_hbm, *rest):
        if mode == "exp":
            ttab_hbm = rest[0]
            rest = rest[1:]
        (out_hbm, acc, svb, tvb, rows, stg, wbuf, sidx, zbuf, stab_v) = rest[:10]
        ttab_v = rest[10] if mode == "exp" else None
        core = lax.axis_index("c")
        sub = lax.axis_index("s")
        wid = core * NS + sub
        _zero_acc(acc, zbuf, wa, sub)
        pltpu.sync_copy(stab_hbm, stab_v)
        if mode == "exp":
            pltpu.sync_copy(ttab_hbm, ttab_v)
        plsc.subcore_barrier()
        base = wid * per_w

        @pl.loop(0, nblk)
        def _(b):
            off = base + b * blk
            pltpu.sync_copy(sv_hbm.at[pl.ds(off, blk)], svb)
            pltpu.sync_copy(tv_hbm.at[pl.ds(off, blk)], tvb)

            @pl.loop(0, blk // 16)
            def _(g):
                s = svb[pl.ds(g * 16, 16)]
                t = tvb[pl.ds(g * 16, 16)]
                s_cl = jnp.minimum(s, N_NODES - 1)
                sidx[...] = jnp.where(s < N_NODES, s, SLOP)
                pltpu.sync_copy(tab_hbm.at[t], rows)
                if mode == "exp":
                    w = jnp.exp(_leaky(plsc.load_gather(stab_v, [s_cl])
                                       + plsc.load_gather(ttab_v, [t])))
                else:
                    w = plsc.load_gather(stab_v, [s_cl])
                wbuf[...] = w
                _scale_rows(rows, stg, wbuf, wg, mode == "exp")
                pltpu.sync_copy(stg, acc.at[sidx], add=True)

        plsc.subcore_barrier()
        _copy_out(acc, out_hbm, core, sub)

    scratch = [
        pltpu.VMEM_SHARED((NA, wa), jnp.float32),
        pltpu.VMEM((blk,), jnp.int32),
        pltpu.VMEM((blk,), jnp.int32),
        pltpu.VMEM((16, wg), jnp.float32),
        pltpu.VMEM((16, wa), jnp.float32),
        pltpu.VMEM((16,), jnp.float32),
        pltpu.VMEM((16,), jnp.int32),
        pltpu.VMEM((16, wa), jnp.float32),
        pltpu.VMEM((nstab,), jnp.float32),
    ]
    args = [sv, tv, table, stab]
    if mode == "exp":
        scratch.append(pltpu.VMEM((nttab,), jnp.float32))
        args.append(ttab)
    fn = pl.kernel(body, mesh=_MESH, compiler_params=_CP,
                   out_type=jax.ShapeDtypeStruct((NC, NA, wa), jnp.float32),
                   scratch_types=scratch)
    out = fn(*args)
    return out[0] + out[1]


def _sc_phase1(sv, dv, nodepack, rf, ep, blk, lcols):
    per_w = ep // NW
    nblk = per_w // blk

    def body(sv_hbm, dv_hbm, np_hbm, rf_hbm, g_hbm, p_hbm,
             svb, dvb, rfb, sblk, dblk, gout, sacc):
        core = lax.axis_index("c")
        sub = lax.axis_index("s")
        wid = core * NS + sub
        base = wid * per_w
        for h in range(H):
            sacc[h, ...] = jnp.zeros((16,), jnp.float32)

        @pl.loop(0, nblk)
        def _(b):
            off = base + b * blk
            pltpu.sync_copy(sv_hbm.at[pl.ds(off, blk)], svb)
            pltpu.sync_copy(dv_hbm.at[pl.ds(off, blk)], dvb)
            pltpu.sync_copy(rf_hbm.at[pl.ds(off, blk)], rfb)

            @pl.loop(0, blk // 16)
            def _(g):
                rows16 = g * 16 + _iota16()
                s = svb[pl.ds(g * 16, 16)]
                d = dvb[pl.ds(g * 16, 16)]
                s_cl = jnp.minimum(s, N_NODES - 1)
                d_cl = jnp.minimum(d, N_NODES - 1)
                pltpu.sync_copy(np_hbm.at[s_cl], sblk)
                pltpu.sync_copy(np_hbm.at[d_cl], dblk)
                valid = s < N_NODES
                for h in range(H):
                    p1 = plsc.load_gather(sblk, [_iota16(), _full16(8 + h)])
                    c1 = plsc.load_gather(sblk, [_iota16(), _full16(12 + h)])
                    p3 = plsc.load_gather(dblk, [_iota16(), _full16(h)])
                    c2 = plsc.load_gather(dblk, [_iota16(), _full16(4 + h)])
                    rfe = plsc.load_gather(rfb, [rows16, _full16(lcols + h)])
                    rfc = plsc.load_gather(rfb, [rows16, _full16(8 + lcols + h)])
                    gh = _leaky(p1 + rfe + p3) * _leaky(c1 + rfc + c2)
                    plsc.store_scatter(gout, [rows16, _full16(h)], gh)
                    sacc[h, ...] += jnp.where(valid, jnp.exp(gh), 0.0)

            pltpu.sync_copy(gout, g_hbm.at[pl.ds(off, blk)])

        pltpu.sync_copy(sacc, p_hbm.at[wid])

    fn = pl.kernel(
        body, mesh=_MESH, compiler_params=_CP,
        out_type=(jax.ShapeDtypeStruct((ep, H), jnp.float32),
                  jax.ShapeDtypeStruct((NW, H, 16), jnp.float32)),
        scratch_types=[
            pltpu.VMEM((blk,), jnp.int32),
            pltpu.VMEM((blk,), jnp.int32),
            pltpu.VMEM((blk, 16), jnp.float32),
            pltpu.VMEM((16, 16), jnp.float32),
            pltpu.VMEM((16, 16), jnp.float32),
            pltpu.VMEM((blk, H), jnp.float32),
            pltpu.VMEM((H, 16), jnp.float32),
        ])
    return fn(sv, dv, nodepack, rf)


def _sc_phase2(sv, dv, xpack, gvals, invs, ep, blk):
    wa = D + 16
    per_w = ep // NW
    nblk = per_w // blk

    def body(sv_hbm, dv_hbm, xp_hbm, g_hbm, invs_hbm,
             out_hbm, acc, svb, dvb, gb, rows, stg, wbuf, sidx, zbuf, invs_v):
        core = lax.axis_index("c")
        sub = lax.axis_index("s")
        wid = core * NS + sub
        _zero_acc(acc, zbuf, wa, sub)
        pltpu.sync_copy(invs_hbm, invs_v)
        plsc.subcore_barrier()
        base = wid * per_w

        @pl.loop(0, nblk)
        def _(b):
            off = base + b * blk
            pltpu.sync_copy(sv_hbm.at[pl.ds(off, blk)], svb)
            pltpu.sync_copy(dv_hbm.at[pl.ds(off, blk)], dvb)
            pltpu.sync_copy(g_hbm.at[pl.ds(off, blk)], gb)

            @pl.loop(0, blk // 16)
            def _(g):
                rows16 = g * 16 + _iota16()
                s = svb[pl.ds(g * 16, 16)]
                d = dvb[pl.ds(g * 16, 16)]
                d_cl = jnp.minimum(d, N_NODES - 1)
                sidx[...] = jnp.where(s < N_NODES, s, SLOP)
                pltpu.sync_copy(xp_hbm.at[d_cl], rows)
                for h in range(H):
                    gh = plsc.load_gather(gb, [rows16, _full16(h)])
                    a1 = jnp.exp(gh) * plsc.load_gather(invs_v, [_full16(h)])
                    wbuf[h, ...] = jnp.exp(a1)
                for i in range(16):
                    for h in range(H):
                        ws = plsc.load_gather(wbuf, [_full16(h), _full16(i)])
                        for c in range(HD // 16):
                            cc = h * HD + c * 16
                            stg[i, pl.ds(cc, 16)] = rows[i, pl.ds(cc, 16)] * ws
                    wrow = plsc.load_gather(
                        wbuf, [jnp.minimum(_iota16(), H - 1), _full16(i)])
                    stg[i, pl.ds(D, 16)] = jnp.where(_iota16() < H, wrow, 0.0)
                pltpu.sync_copy(stg, acc.at[sidx], add=True)

        plsc.subcore_barrier()
        _copy_out(acc, out_hbm, core, sub)

    fn = pl.kernel(
        body, mesh=_MESH, compiler_params=_CP,
        out_type=jax.ShapeDtypeStruct((NC, NA, wa), jnp.float32),
        scratch_types=[
            pltpu.VMEM_SHARED((NA, wa), jnp.float32),
            pltpu.VMEM((blk,), jnp.int32),
            pltpu.VMEM((blk,), jnp.int32),
            pltpu.VMEM((blk, H), jnp.float32),
            pltpu.VMEM((16, D), jnp.float32),
            pltpu.VMEM((16, wa), jnp.float32),
            pltpu.VMEM((H, 16), jnp.float32),
            pltpu.VMEM((16,), jnp.int32),
            pltpu.VMEM((16, wa), jnp.float32),
            pltpu.VMEM((16,), jnp.float32),
        ])
    out = fn(sv, dv, xpack, gvals, invs)
    return out[0] + out[1]


def _extract_i32(vec, j):
    return lax.reduce_max(
        jnp.where(_iota16() == j, vec, jnp.int32(-2**31 + 1)), axes=(0,))


def _sc_rf(idxv, er1, rsc16):
    K = E_EDGES // 64
    WIN = K + 16
    NG = K // 16

    def body(idx_hbm, er_hbm, rsc_hbm, out_hbm, bacc_hbm,
             idxb, erb, win, rsc_v, prevb, zrow):
        core = lax.axis_index("c")
        sub = lax.axis_index("s")
        wid = core * NS + sub
        pltpu.sync_copy(rsc_hbm, rsc_v)
        zrow[0, ...] = jnp.zeros((16,), jnp.float32)
        for cc in range(2):
            c = wid * 2 + cc
            base = c * K

            @pl.loop(0, WIN)
            def _(r):
                win[r, ...] = jnp.zeros((16,), jnp.float32)

            pltpu.sync_copy(idx_hbm.at[pl.ds(base, K)], idxb)
            pltpu.sync_copy(er_hbm.at[pl.ds(base, K)], erb)
            prevb[...] = _full16(-1)

            @pl.when(c > 0)
            def _():
                pltpu.sync_copy(idx_hbm.at[pl.ds(base - 16, 16)], prevb)

            prev = _extract_i32(prevb[...], 15)
            ubase = _extract_i32(idxb[pl.ds(0, 16)], 0)
            ulast = _extract_i32(idxb[pl.ds(K - 16, 16)], 15)

            def do_edges(uv, rv, lanes):
                urel = uv - ubase
                for i in lanes:
                    u_rel = _extract_i32(urel, i)
                    r = _extract_i32(rv, i)
                    plsc.addupdate(win.at[u_rel], rsc_v[r, ...])

            @pl.loop(0, NG)
            def _(g):
                do_edges(idxb[pl.ds(g * 16, 16)], erb[pl.ds(g * 16, 16)],
                         range(16))

            do_edges(idxb[pl.ds(K - 16, 16)], erb[pl.ds(K - 16, 16)],
                     range(8, 16))

            fresh = (prev != ubase).astype(jnp.int32)
            start = 1 - fresh
            count = ulast - ubase - start + 1
            nck = count // 16
            rem = count - nck * 16

            @pl.loop(0, nck)
            def _(k):
                pltpu.sync_copy(
                    win.at[pl.ds(start + k * 16, 16)],
                    out_hbm.at[pl.ds(ubase + start + k * 16, 16)])

            @pl.loop(0, rem)
            def _(k):
                pltpu.sync_copy(
                    win.at[pl.ds(start + nck * 16 + k, 1)],
                    out_hbm.at[pl.ds(ubase + start + nck * 16 + k, 1)])

            @pl.when(fresh == 0)
            def _():
                pltpu.sync_copy(win.at[pl.ds(0, 1)],
                                bacc_hbm.at[pl.ds(c, 1)])

            @pl.when(fresh == 1)
            def _():
                pltpu.sync_copy(zrow, bacc_hbm.at[pl.ds(c, 1)])

    fn = pl.kernel(
        body, mesh=_MESH, compiler_params=_CP,
        out_type=(jax.ShapeDtypeStruct((E_EDGES, 16), jnp.float32),
                  jax.ShapeDtypeStruct((64, 16), jnp.float32)),
        scratch_types=[
            pltpu.VMEM((K,), jnp.int32),
            pltpu.VMEM((K,), jnp.int32),
            pltpu.VMEM((WIN, 16), jnp.float32),
            pltpu.VMEM((R_RELS, 16), jnp.float32),
            pltpu.VMEM((16,), jnp.int32),
            pltpu.VMEM((1, 16), jnp.float32),
        ])
    return fn(idxv, er1, rsc16)


def _sort_pairs(a, mult):
    key = a[:, 0] * mult + a[:, 1]
    return a[jnp.argsort(key)]


def _unique_sorted(a_sorted, mult, oob):
    key = a_sorted[:, 0] * mult + a_sorted[:, 1]
    n = key.shape[0]
    valid = jnp.concatenate([jnp.ones((1,), bool), key[1:] != key[:-1]])
    inv = jnp.cumsum(valid) - 1
    uk = jnp.full((n,), oob * mult, key.dtype).at[inv].set(key)
    rows = jnp.stack([uk // mult, uk % mult], axis=1)
    return rows, inv


def kernel(ent_emb, rel_emb, attr_emb, wr_w, wr_b, wa_w, wa_b,
           ent_attn_w, ent_attn_b, concept_attn_w, concept_attn_b,
           all_matix, attr_matrix):
    n = ent_emb.shape[0]

    er = _sort_pairs(all_matix[:, 3:5], R_RELS)
    rel_index, _ = _unique_sorted(er, R_RELS, n)
    ea = _sort_pairs(attr_matrix[:, 0:2], A_ATTRS)
    attr_index, _ = _unique_sorted(ea, A_ATTRS, n)
    ee = _sort_pairs(all_matix[:, 0:2], n)
    index, idx = _unique_sorted(ee, n, n)
    src, dst = index[:, 0], index[:, 1]

    se = _matmul(ent_emb, wr_w[:D]).reshape(-1) + wr_b[0]
    sr = _matmul(rel_emb, wr_w[D:]).reshape(-1)
    sa_e = _matmul(ent_emb, wa_w[:D]).reshape(-1) + wa_b[0]
    sa = _matmul(attr_emb, wa_w[D:]).reshape(-1)

    w_rf_e = jnp.stack([ent_attn_w[l, h, HD:HD + RD, 0]
                        for l in range(L) for h in range(H)], axis=1)
    w_rf_c = jnp.stack([concept_attn_w[l, h, RD + AD:RD + AD + RD, 0]
                        for l in range(L) for h in range(H)], axis=1)
    rsc16 = _matmul(rel_emb, jnp.concatenate([w_rf_e, w_rf_c], axis=1))

    acc_a = _sc_soft_agg(rel_index[:, 0], rel_index[:, 1], rel_emb,
                         se, sr, E_EDGES, 2000, "exp")
    concept_rel = jax.nn.relu(
        jnp.where(acc_a[:n, RD:RD + 1] > 0,
                  acc_a[:n, :RD] / acc_a[:n, RD:RD + 1], 0.0))

    ep_b = 204800
    pad_b = ep_b - EA_EDGES
    bi = jnp.concatenate([attr_index[:, 0], jnp.full((pad_b,), n, jnp.int32)])
    bt = jnp.concatenate([attr_index[:, 1], jnp.zeros((pad_b,), jnp.int32)])
    acc_b = _sc_soft_agg(bi, bt, attr_emb, sa_e, sa, ep_b, 1600, "exp")
    concept_attr = jax.nn.relu(
        jnp.where(acc_b[:n, AD:AD + 1] > 0,
                  acc_b[:n, :AD] / acc_b[:n, AD:AD + 1], 0.0))

    bounds = jnp.searchsorted(src, jnp.arange(n + 1, dtype=src.dtype))
    deg = (bounds[1:] - bounds[:-1]).astype(jnp.float32)
    invdeg = jnp.where(deg > 0, 1.0 / deg, 0.0)
    acc_x = _sc_soft_agg(src, jnp.minimum(dst, n - 1), ent_emb,
                         invdeg, None, E_EDGES, 2000, "lin")
    x = acc_x[:n]

    rf_sum, bacc = _sc_rf(idx.astype(jnp.int32), er[:, 1].astype(jnp.int32),
                          rsc16)
    ubases = idx[jnp.arange(64) * (E_EDGES // 64)]
    rf_sum = rf_sum.at[ubases].add(bacc)
    ss = jnp.searchsorted(idx, jnp.arange(E_EDGES + 1, dtype=idx.dtype))
    cnt = (ss[1:] - ss[:-1]).astype(jnp.float32)
    rf = rf_sum * jnp.where(cnt > 0, 1.0 / cnt, 0.0)[:, None]

    concept_cat = jnp.concatenate([concept_rel, concept_attr], axis=1)
    w_c1 = jnp.stack([concept_attn_w[l, h, :RD + AD, 0]
                      for l in range(L) for h in range(H)], axis=1)
    w_c2 = jnp.stack([concept_attn_w[l, h, RD + AD + RD:, 0]
                      for l in range(L) for h in range(H)], axis=1)
    c1 = _matmul(concept_cat, w_c1) + concept_attn_b.reshape(1, L * H)
    c2 = _matmul(concept_cat, w_c2)

    outputs = []
    for l in range(L):
        xr = jax.nn.relu(x)
        w_p1 = jnp.stack([ent_attn_w[l, h, :HD, 0] for h in range(H)], axis=1)
        w_p3 = jnp.stack([ent_attn_w[l, h, HD + RD:, 0] for h in range(H)],
                         axis=1)
        p1 = jnp.stack(
            [xr[:, h * HD:(h + 1) * HD] @ w_p1[:, h] for h in range(H)],
            axis=1) + ent_attn_b[l, :, 0][None, :]
        p3 = jnp.stack(
            [xr[:, h * HD:(h + 1) * HD] @ w_p3[:, h] for h in range(H)],
            axis=1)
        nodepack = jnp.concatenate(
            [p3, c2[:, l * H:(l + 1) * H], p1, c1[:, l * H:(l + 1) * H]],
            axis=1)

        gvals, parts = _sc_phase1(src, jnp.minimum(dst, n - 1), nodepack, rf,
                                  E_EDGES, 2000, l * H)
        s_glob = jnp.sum(parts, axis=(0, 2))
        invs = jnp.concatenate([1.0 / s_glob,
                                jnp.zeros((16 - H,), jnp.float32)])
        acc2 = _sc_phase2(src, jnp.minimum(dst, n - 1), xr, gvals, invs,
                          E_EDGES, 2000)
        num = acc2[:n, :D]
        den = acc2[:n, D:D + H]
        den_rep = jnp.repeat(den, HD, axis=1)
        x = jnp.tanh(jnp.where(den_rep > 0, num / den_rep, 0.0))
        outputs.append(x)
    return jnp.concatenate(outputs, axis=1)


# speedup vs baseline: 29.7022x; 4.0269x over previous
import dataclasses
import functools
import jax
import jax.numpy as jnp
from jax import lax
from jax.experimental import pallas as pl
from jax.experimental.pallas import tpu as pltpu
from jax.experimental.pallas import tpu_sc as plsc

N_NODES, E_EDGES, R_RELS, A_ATTRS, EA_EDGES = 10000, 320000, 500, 2000, 200000
D, RD, AD = 128, 64, 64
L, H = 2, 4
HD = D // H

NC, NS, LN = 2, 16, 16
NW = NC * NS
NA = 10240
SLOP = N_NODES


def _leaky(x):
    return jnp.where(x >= 0, x, 0.3 * x)


def _iota16():
    return lax.iota(jnp.int32, 16)


def _full16(v):
    return jnp.full((16,), v, jnp.int32)


def _matmul_kernel(x_ref, w_ref, o_ref):
    o_ref[...] = jnp.dot(x_ref[...], w_ref[...],
                         preferred_element_type=jnp.float32)


def _matmul(x, w, block_rows=1024):
    n, k = x.shape
    m = w.shape[1]
    n_pad = ((n + block_rows - 1) // block_rows) * block_rows
    if n_pad != n:
        x = jnp.pad(x, ((0, n_pad - n), (0, 0)))
    out = pl.pallas_call(
        _matmul_kernel,
        grid=(n_pad // block_rows,),
        in_specs=[pl.BlockSpec((block_rows, k), lambda i: (i, 0)),
                  pl.BlockSpec((k, m), lambda i: (0, 0))],
        out_specs=pl.BlockSpec((block_rows, m), lambda i: (i, 0)),
        out_shape=jax.ShapeDtypeStruct((n_pad, m), jnp.float32),
    )(x, w)
    return out[:n]


_MESH = plsc.VectorSubcoreMesh(core_axis_name="c", subcore_axis_name="s")
_CP = pltpu.CompilerParams()
if "needs_layout_passes" in pltpu.CompilerParams.__dataclass_fields__:
    _CP = dataclasses.replace(_CP, needs_layout_passes=False,
                              use_tc_tiling_on_sc=False)


def _zero_acc(acc, zbuf, wa, sub):
    for c in range(wa // 16):
        zbuf[0, pl.ds(c * 16, 16)] = jnp.zeros((16,), jnp.float32)
    for r in range(1, 16):
        for c in range(wa // 16):
            zbuf[r, pl.ds(c * 16, 16)] = zbuf[0, pl.ds(c * 16, 16)]
    stripe = NA // NS

    @pl.loop(0, stripe // 16)
    def _(k):
        pltpu.sync_copy(zbuf, acc.at[pl.ds(sub * stripe + k * 16, 16)])


def _copy_out(acc, out, core, sub):
    stripe = NA // NS
    pltpu.sync_copy(acc.at[pl.ds(sub * stripe, stripe)],
                    out.at[core, pl.ds(sub * stripe, stripe)])


def _scale_rows(rows, stg, wbuf, wg, wcol):
    for i in range(16):
        ws = plsc.load_gather(wbuf, [_full16(i)])
        for c in range(wg // 16):
            stg[i, pl.ds(c * 16, 16)] = rows[i, pl.ds(c * 16, 16)] * ws
        if wcol:
            stg[i, pl.ds(wg, 16)] = jnp.where(_iota16() == 0, ws, 0.0)


def _sc_soft_agg(sv, tv, table, stab, ttab, ep, blk, mode):
    wg = table.shape[1]
    wa = wg + 16 if mode == "exp" else wg
    per_w = ep // NW
    nblk = per_w // blk
    nstab = stab.shape[0]
    nttab = ttab.shape[0] if ttab is not None else 0

    def body(sv_hbm, tv_hbm, tab_hbm, stab_hbm, *rest):
        if mode == "exp":
            ttab_hbm = rest[0]
            rest = rest[1:]
        (out_hbm, acc, svb, tvb, rows, stg, wbuf, sidx, zbuf, stab_v) = rest[:10]
        ttab_v = rest[10] if mode == "exp" else None
        core = lax.axis_index("c")
        sub = lax.axis_index("s")
        wid = core * NS + sub
        _zero_acc(acc, zbuf, wa, sub)
        pltpu.sync_copy(stab_hbm, stab_v)
        if mode == "exp":
            pltpu.sync_copy(ttab_hbm, ttab_v)
        plsc.subcore_barrier()
        base = wid * per_w

        @pl.loop(0, nblk)
        def _(b):
            off = base + b * blk
            pltpu.sync_copy(sv_hbm.at[pl.ds(off, blk)], svb)
            pltpu.sync_copy(tv_hbm.at[pl.ds(off, blk)], tvb)

            @pl.loop(0, blk // 16)
            def _(g):
                s = svb[pl.ds(g * 16, 16)]
                t = tvb[pl.ds(g * 16, 16)]
                s_cl = jnp.minimum(s, N_NODES - 1)
                sidx[...] = jnp.where(s < N_NODES, s, SLOP)
                pltpu.sync_copy(tab_hbm.at[t], rows)
                if mode == "exp":
                    w = jnp.exp(_leaky(plsc.load_gather(stab_v, [s_cl])
                                       + plsc.load_gather(ttab_v, [t])))
                else:
                    w = plsc.load_gather(stab_v, [s_cl])
                wbuf[...] = w
                _scale_rows(rows, stg, wbuf, wg, mode == "exp")
                pltpu.sync_copy(stg, acc.at[sidx], add=True)

        plsc.subcore_barrier()
        _copy_out(acc, out_hbm, core, sub)

    scratch = [
        pltpu.VMEM_SHARED((NA, wa), jnp.float32),
        pltpu.VMEM((blk,), jnp.int32),
        pltpu.VMEM((blk,), jnp.int32),
        pltpu.VMEM((16, wg), jnp.float32),
        pltpu.VMEM((16, wa), jnp.float32),
        pltpu.VMEM((16,), jnp.float32),
        pltpu.VMEM((16,), jnp.int32),
        pltpu.VMEM((16, wa), jnp.float32),
        pltpu.VMEM((nstab,), jnp.float32),
    ]
    args = [sv, tv, table, stab]
    if mode == "exp":
        scratch.append(pltpu.VMEM((nttab,), jnp.float32))
        args.append(ttab)
    fn = pl.kernel(body, mesh=_MESH, compiler_params=_CP,
                   out_type=jax.ShapeDtypeStruct((NC, NA, wa), jnp.float32),
                   scratch_types=scratch)
    out = fn(*args)
    return out[0] + out[1]


def _sc_phase1(sv, dv, nodepack, rf, ep, blk, lcols):
    per_w = ep // NW
    nblk = per_w // blk

    def body(sv_hbm, dv_hbm, np_hbm, rf_hbm, g_hbm, p_hbm,
             svb, dvb, rfb, sblk, dblk, gout, sacc):
        core = lax.axis_index("c")
        sub = lax.axis_index("s")
        wid = core * NS + sub
        base = wid * per_w
        for h in range(H):
            sacc[h, ...] = jnp.zeros((16,), jnp.float32)

        @pl.loop(0, nblk)
        def _(b):
            off = base + b * blk
            pltpu.sync_copy(sv_hbm.at[pl.ds(off, blk)], svb)
            pltpu.sync_copy(dv_hbm.at[pl.ds(off, blk)], dvb)
            pltpu.sync_copy(rf_hbm.at[pl.ds(off, blk)], rfb)

            @pl.loop(0, blk // 16)
            def _(g):
                rows16 = g * 16 + _iota16()
                s = svb[pl.ds(g * 16, 16)]
                d = dvb[pl.ds(g * 16, 16)]
                s_cl = jnp.minimum(s, N_NODES - 1)
                d_cl = jnp.minimum(d, N_NODES - 1)
                pltpu.sync_copy(np_hbm.at[s_cl], sblk)
                pltpu.sync_copy(np_hbm.at[d_cl], dblk)
                valid = s < N_NODES
                for h in range(H):
                    p1 = plsc.load_gather(sblk, [_iota16(), _full16(8 + h)])
                    c1 = plsc.load_gather(sblk, [_iota16(), _full16(12 + h)])
                    p3 = plsc.load_gather(dblk, [_iota16(), _full16(h)])
                    c2 = plsc.load_gather(dblk, [_iota16(), _full16(4 + h)])
                    rfe = plsc.load_gather(rfb, [rows16, _full16(lcols + h)])
                    rfc = plsc.load_gather(rfb, [rows16, _full16(8 + lcols + h)])
                    gh = _leaky(p1 + rfe + p3) * _leaky(c1 + rfc + c2)
                    plsc.store_scatter(gout, [rows16, _full16(h)], gh)
                    sacc[h, ...] += jnp.where(valid, jnp.exp(gh), 0.0)

            pltpu.sync_copy(gout, g_hbm.at[pl.ds(off, blk)])

        pltpu.sync_copy(sacc, p_hbm.at[wid])

    fn = pl.kernel(
        body, mesh=_MESH, compiler_params=_CP,
        out_type=(jax.ShapeDtypeStruct((ep, H), jnp.float32),
                  jax.ShapeDtypeStruct((NW, H, 16), jnp.float32)),
        scratch_types=[
            pltpu.VMEM((blk,), jnp.int32),
            pltpu.VMEM((blk,), jnp.int32),
            pltpu.VMEM((blk, 16), jnp.float32),
            pltpu.VMEM((16, 16), jnp.float32),
            pltpu.VMEM((16, 16), jnp.float32),
            pltpu.VMEM((blk, H), jnp.float32),
            pltpu.VMEM((H, 16), jnp.float32),
        ])
    return fn(sv, dv, nodepack, rf)


def _sc_phase2(sv, dv, xpack, gvals, invs, ep, blk):
    wa = D + 16
    per_w = ep // NW
    nblk = per_w // blk

    def body(sv_hbm, dv_hbm, xp_hbm, g_hbm, invs_hbm,
             out_hbm, acc, svb, dvb, gb, rows, stg, wbuf, sidx, zbuf, invs_v):
        core = lax.axis_index("c")
        sub = lax.axis_index("s")
        wid = core * NS + sub
        _zero_acc(acc, zbuf, wa, sub)
        pltpu.sync_copy(invs_hbm, invs_v)
        plsc.subcore_barrier()
        base = wid * per_w

        @pl.loop(0, nblk)
        def _(b):
            off = base + b * blk
            pltpu.sync_copy(sv_hbm.at[pl.ds(off, blk)], svb)
            pltpu.sync_copy(dv_hbm.at[pl.ds(off, blk)], dvb)
            pltpu.sync_copy(g_hbm.at[pl.ds(off, blk)], gb)

            @pl.loop(0, blk // 16)
            def _(g):
                rows16 = g * 16 + _iota16()
                s = svb[pl.ds(g * 16, 16)]
                d = dvb[pl.ds(g * 16, 16)]
                d_cl = jnp.minimum(d, N_NODES - 1)
                sidx[...] = jnp.where(s < N_NODES, s, SLOP)
                pltpu.sync_copy(xp_hbm.at[d_cl], rows)
                for h in range(H):
                    gh = plsc.load_gather(gb, [rows16, _full16(h)])
                    a1 = jnp.exp(gh) * plsc.load_gather(invs_v, [_full16(h)])
                    wbuf[h, ...] = jnp.exp(a1)
                for i in range(16):
                    for h in range(H):
                        ws = plsc.load_gather(wbuf, [_full16(h), _full16(i)])
                        for c in range(HD // 16):
                            cc = h * HD + c * 16
                            stg[i, pl.ds(cc, 16)] = rows[i, pl.ds(cc, 16)] * ws
                    wrow = plsc.load_gather(
                        wbuf, [jnp.minimum(_iota16(), H - 1), _full16(i)])
                    stg[i, pl.ds(D, 16)] = jnp.where(_iota16() < H, wrow, 0.0)
                pltpu.sync_copy(stg, acc.at[sidx], add=True)

        plsc.subcore_barrier()
        _copy_out(acc, out_hbm, core, sub)

    fn = pl.kernel(
        body, mesh=_MESH, compiler_params=_CP,
        out_type=jax.ShapeDtypeStruct((NC, NA, wa), jnp.float32),
        scratch_types=[
            pltpu.VMEM_SHARED((NA, wa), jnp.float32),
            pltpu.VMEM((blk,), jnp.int32),
            pltpu.VMEM((blk,), jnp.int32),
            pltpu.VMEM((blk, H), jnp.float32),
            pltpu.VMEM((16, D), jnp.float32),
            pltpu.VMEM((16, wa), jnp.float32),
            pltpu.VMEM((H, 16), jnp.float32),
            pltpu.VMEM((16,), jnp.int32),
            pltpu.VMEM((16, wa), jnp.float32),
            pltpu.VMEM((16,), jnp.float32),
        ])
    out = fn(sv, dv, xpack, gvals, invs)
    return out[0] + out[1]


def _extract_i32(vec, j):
    return lax.reduce_max(
        jnp.where(_iota16() == j, vec, jnp.int32(-2**31 + 1)), axes=(0,))


def _sc_rf(idxv, er1, rsc16):
    K = E_EDGES // 64
    WIN = K + 16
    NG = K // 16

    def body(idx_hbm, er_hbm, rsc_hbm, out_hbm, bacc_hbm,
             idxb, erb, win, rsc_v, prevb, zrow):
        core = lax.axis_index("c")
        sub = lax.axis_index("s")
        wid = core * NS + sub
        pltpu.sync_copy(rsc_hbm, rsc_v)
        zrow[0, ...] = jnp.zeros((16,), jnp.float32)
        for cc in range(2):
            c = wid * 2 + cc
            base = c * K

            @pl.loop(0, WIN)
            def _(r):
                win[r, ...] = jnp.zeros((16,), jnp.float32)

            pltpu.sync_copy(idx_hbm.at[pl.ds(base, K)], idxb)
            pltpu.sync_copy(er_hbm.at[pl.ds(base, K)], erb)
            prevb[...] = _full16(-1)

            @pl.when(c > 0)
            def _():
                pltpu.sync_copy(idx_hbm.at[pl.ds(base - 16, 16)], prevb)

            prev = _extract_i32(prevb[...], 15)
            ubase = _extract_i32(idxb[pl.ds(0, 16)], 0)
            ulast = _extract_i32(idxb[pl.ds(K - 16, 16)], 15)

            def do_edges(uv, rv, lanes):
                urel = uv - ubase
                for i in lanes:
                    u_rel = _extract_i32(urel, i)
                    r = _extract_i32(rv, i)
                    plsc.addupdate(win.at[u_rel], rsc_v[r, ...])

            @pl.loop(0, NG)
            def _(g):
                do_edges(idxb[pl.ds(g * 16, 16)], erb[pl.ds(g * 16, 16)],
                         range(16))

            do_edges(idxb[pl.ds(K - 16, 16)], erb[pl.ds(K - 16, 16)],
                     range(8, 16))

            fresh = (prev != ubase).astype(jnp.int32)
            start = 1 - fresh
            count = ulast - ubase - start + 1
            nck = count // 16
            rem = count - nck * 16

            @pl.loop(0, nck)
            def _(k):
                pltpu.sync_copy(
                    win.at[pl.ds(start + k * 16, 16)],
                    out_hbm.at[pl.ds(ubase + start + k * 16, 16)])

            @pl.loop(0, rem)
            def _(k):
                pltpu.sync_copy(
                    win.at[pl.ds(start + nck * 16 + k, 1)],
                    out_hbm.at[pl.ds(ubase + start + nck * 16 + k, 1)])

            @pl.when(fresh == 0)
            def _():
                pltpu.sync_copy(win.at[pl.ds(0, 1)],
                                bacc_hbm.at[pl.ds(c, 1)])

            @pl.when(fresh == 1)
            def _():
                pltpu.sync_copy(zrow, bacc_hbm.at[pl.ds(c, 1)])

    fn = pl.kernel(
        body, mesh=_MESH, compiler_params=_CP,
        out_type=(jax.ShapeDtypeStruct((E_EDGES, 16), jnp.float32),
                  jax.ShapeDtypeStruct((64, 16), jnp.float32)),
        scratch_types=[
            pltpu.VMEM((K,), jnp.int32),
            pltpu.VMEM((K,), jnp.int32),
            pltpu.VMEM((WIN, 16), jnp.float32),
            pltpu.VMEM((R_RELS, 16), jnp.float32),
            pltpu.VMEM((16,), jnp.int32),
            pltpu.VMEM((1, 16), jnp.float32),
        ])
    return fn(idxv, er1, rsc16)


def _sort_pairs(a, mult):
    key = jnp.sort(a[:, 0] * mult + a[:, 1])
    return jnp.stack([key // mult, key % mult], axis=1)


def _unique_sorted(a_sorted, mult, oob):
    key = a_sorted[:, 0] * mult + a_sorted[:, 1]
    n = key.shape[0]
    valid = jnp.concatenate([jnp.ones((1,), bool), key[1:] != key[:-1]])
    inv = jnp.cumsum(valid) - 1
    uk = jnp.full((n,), oob * mult, key.dtype).at[inv].set(key)
    rows = jnp.stack([uk // mult, uk % mult], axis=1)
    pos = jnp.full((n + 1,), n, key.dtype).at[inv].min(
        jnp.arange(n, dtype=key.dtype))
    return rows, inv, pos


def kernel(ent_emb, rel_emb, attr_emb, wr_w, wr_b, wa_w, wa_b,
           ent_attn_w, ent_attn_b, concept_attn_w, concept_attn_b,
           all_matix, attr_matrix):
    n = ent_emb.shape[0]

    er = _sort_pairs(all_matix[:, 3:5], R_RELS)
    rel_index, _, _ = _unique_sorted(er, R_RELS, n)
    ea = _sort_pairs(attr_matrix[:, 0:2], A_ATTRS)
    attr_index, _, _ = _unique_sorted(ea, A_ATTRS, n)
    ee = _sort_pairs(all_matix[:, 0:2], n)
    index, idx, pos = _unique_sorted(ee, n, n)
    src, dst = index[:, 0], index[:, 1]

    se = _matmul(ent_emb, wr_w[:D]).reshape(-1) + wr_b[0]
    sr = _matmul(rel_emb, wr_w[D:]).reshape(-1)
    sa_e = _matmul(ent_emb, wa_w[:D]).reshape(-1) + wa_b[0]
    sa = _matmul(attr_emb, wa_w[D:]).reshape(-1)

    w_rf_e = jnp.stack([ent_attn_w[l, h, HD:HD + RD, 0]
                        for l in range(L) for h in range(H)], axis=1)
    w_rf_c = jnp.stack([concept_attn_w[l, h, RD + AD:RD + AD + RD, 0]
                        for l in range(L) for h in range(H)], axis=1)
    rsc16 = _matmul(rel_emb, jnp.concatenate([w_rf_e, w_rf_c], axis=1))

    acc_a = _sc_soft_agg(rel_index[:, 0], rel_index[:, 1], rel_emb,
                         se, sr, E_EDGES, 2000, "exp")
    concept_rel = jax.nn.relu(
        jnp.where(acc_a[:n, RD:RD + 1] > 0,
                  acc_a[:n, :RD] / acc_a[:n, RD:RD + 1], 0.0))

    ep_b = 204800
    pad_b = ep_b - EA_EDGES
    bi = jnp.concatenate([attr_index[:, 0], jnp.full((pad_b,), n, jnp.int32)])
    bt = jnp.concatenate([attr_index[:, 1], jnp.zeros((pad_b,), jnp.int32)])
    acc_b = _sc_soft_agg(bi, bt, attr_emb, sa_e, sa, ep_b, 1600, "exp")
    concept_attr = jax.nn.relu(
        jnp.where(acc_b[:n, AD:AD + 1] > 0,
                  acc_b[:n, :AD] / acc_b[:n, AD:AD + 1], 0.0))

    bounds = jnp.searchsorted(src, jnp.arange(n + 1, dtype=src.dtype))
    deg = (bounds[1:] - bounds[:-1]).astype(jnp.float32)
    invdeg = jnp.where(deg > 0, 1.0 / deg, 0.0)
    acc_x = _sc_soft_agg(src, jnp.minimum(dst, n - 1), ent_emb,
                         invdeg, None, E_EDGES, 2000, "lin")
    x = acc_x[:n]

    rf_sum, bacc = _sc_rf(idx.astype(jnp.int32), er[:, 1].astype(jnp.int32),
                          rsc16)
    ubases = idx[jnp.arange(64) * (E_EDGES // 64)]
    rf_sum = rf_sum.at[ubases].add(bacc)
    cnt = (pos[1:] - pos[:-1]).astype(jnp.float32)
    rf = rf_sum * jnp.where(cnt > 0, 1.0 / cnt, 0.0)[:, None]

    concept_cat = jnp.concatenate([concept_rel, concept_attr], axis=1)
    w_c1 = jnp.stack([concept_attn_w[l, h, :RD + AD, 0]
                      for l in range(L) for h in range(H)], axis=1)
    w_c2 = jnp.stack([concept_attn_w[l, h, RD + AD + RD:, 0]
                      for l in range(L) for h in range(H)], axis=1)
    c1 = _matmul(concept_cat, w_c1) + concept_attn_b.reshape(1, L * H)
    c2 = _matmul(concept_cat, w_c2)

    outputs = []
    for l in range(L):
        xr = jax.nn.relu(x)
        w_p1 = jnp.stack([ent_attn_w[l, h, :HD, 0] for h in range(H)], axis=1)
        w_p3 = jnp.stack([ent_attn_w[l, h, HD + RD:, 0] for h in range(H)],
                         axis=1)
        p1 = jnp.stack(
            [xr[:, h * HD:(h + 1) * HD] @ w_p1[:, h] for h in range(H)],
            axis=1) + ent_attn_b[l, :, 0][None, :]
        p3 = jnp.stack(
            [xr[:, h * HD:(h + 1) * HD] @ w_p3[:, h] for h in range(H)],
            axis=1)
        nodepack = jnp.concatenate(
            [p3, c2[:, l * H:(l + 1) * H], p1, c1[:, l * H:(l + 1) * H]],
            axis=1)

        gvals, parts = _sc_phase1(src, jnp.minimum(dst, n - 1), nodepack, rf,
                                  E_EDGES, 2000, l * H)
        s_glob = jnp.sum(parts, axis=(0, 2))
        invs = jnp.concatenate([1.0 / s_glob,
                                jnp.zeros((16 - H,), jnp.float32)])
        acc2 = _sc_phase2(src, jnp.minimum(dst, n - 1), xr, gvals, invs,
                          E_EDGES, 2000)
        num = acc2[:n, :D]
        den = acc2[:n, D:D + H]
        den_rep = jnp.repeat(den, HD, axis=1)
        x = jnp.tanh(jnp.where(den_rep > 0, num / den_rep, 0.0))
        outputs.append(x)
    return jnp.concatenate(outputs, axis=1)
